# Initial kernel scaffold; baseline (speedup 1.0000x reference)
import jax
import jax.numpy as jnp
from jax.experimental import pallas as pl


def kernel(x, edge_attr, conv0_Wl, conv0_bl, conv0_Wr, pool0_Wrel, pool0_brel, pool0_Wroot, conv1_Wl, conv1_bl, conv1_Wr, pool1_Wrel, pool1_brel, pool1_Wroot, conv2_Wl, conv2_bl, conv2_Wr, pool2_Wrel, pool2_brel, pool2_Wroot, enc_W1, enc_b1, enc_W2, enc_b2, grade_W, grade_b, haz_W, haz_b, edge_index, batch, pat_idxs):
    raise NotImplementedError("write your pallas kernel here")


# speedup vs baseline: 10.8697x; 10.8697x over previous
import functools

import jax
import jax.numpy as jnp
from jax import lax
from jax.experimental import pallas as pl
from jax.experimental.pallas import tpu as pltpu
from jax.experimental.pallas import tpu_sc as plsc

NNODES = 10000
NP = 10240
NE = 320000
HD = 128
WA = 144
NC = 2
NS = 16
NW = NC * NS
CH = 128
EP = ((NE + NW * CH - 1) // (NW * CH)) * (NW * CH)
CPW = EP // (NW * CH)
RPT = NP // NS
NEG_INF = float("-inf")


@functools.cache
def _make_segsum(width):
  mesh = plsc.VectorSubcoreMesh(core_axis_name="c", subcore_axis_name="s")

  def body(table, src, dst, out, srcb, dstb, rowsb, acc, gsem):
    cid = lax.axis_index("c")
    sid = lax.axis_index("s")
    wid = cid * NS + sid
    zero16 = jnp.zeros((16,), jnp.float32)

    def zrow(i, carry):
      for j in range(width // 16):
        rowsb[i, pl.ds(j * 16, 16)] = zero16
      return carry
    lax.fori_loop(0, CH, zrow, 0, unroll=False)
    for j in range(RPT // CH):
      pltpu.sync_copy(rowsb, acc.at[pl.ds(sid * RPT + j * CH, CH)])
    plsc.subcore_barrier()

    def step(i, carry):
      base = (wid * CPW + i) * CH
      pltpu.sync_copy(src.at[pl.ds(base, CH)], srcb.at[0])
      pltpu.sync_copy(dst.at[pl.ds(base, CH)], dstb.at[0])
      pltpu.async_copy(table.at[srcb.at[0]], rowsb, gsem).wait()
      pltpu.sync_copy(rowsb, acc.at[dstb.at[0]], add=True)
      return carry
    lax.fori_loop(0, CPW, step, 0, unroll=False)
    plsc.subcore_barrier()
    pltpu.sync_copy(acc.at[pl.ds(sid * RPT, RPT)],
                    out.at[cid, pl.ds(sid * RPT, RPT)])

  return pl.kernel(
      body,
      out_type=jax.ShapeDtypeStruct((NC, NP, width), jnp.float32),
      mesh=mesh,
      scratch_types=[
          pltpu.VMEM((1, CH), jnp.int32),
          pltpu.VMEM((1, CH), jnp.int32),
          pltpu.VMEM((CH, width), jnp.float32),
          pltpu.VMEM_SHARED((NP, width), jnp.float32),
          pltpu.SemaphoreType.DMA,
      ],
      compiler_params=pltpu.CompilerParams(use_tc_tiling_on_sc=False),
      name=f"segsum{width}",
  )


def _segsum_aug(table, src, dst):
  return _make_segsum(WA)(table, src, dst)


def _segsum_feat(table, src, dst):
  return _make_segsum(HD)(table, src, dst)


def _norm_body(x_ref, out_ref):
  x = x_ref[...]
  m = jnp.max(x, axis=0, keepdims=True)
  col = lax.broadcasted_iota(jnp.int32, (1, HD), 1)
  xn = jnp.where(col < 12, x / m, x)
  out_ref[...] = jnp.zeros((NP, WA), jnp.float32)
  out_ref[0:NNODES, 0:HD] = xn
  col2 = lax.broadcasted_iota(jnp.int32, (NNODES, WA - HD), 1)
  out_ref[0:NNODES, HD:WA] = jnp.where(col2 == 0, 1.0, 0.0)


def _normalize(x):
  return pl.pallas_call(
      _norm_body,
      out_shape=jax.ShapeDtypeStruct((NP, WA), jnp.float32),
      name="normalize",
  )(x)


_BR = 1024


def _conv_body(parts_ref, xa_ref, wl_ref, bl_ref, wr_ref, out_ref):
  num = parts_ref[0, :, 0:HD] + parts_ref[1, :, 0:HD]
  cnt = parts_ref[0, :, HD:HD + 1] + parts_ref[1, :, HD:HD + 1]
  x = xa_ref[:, 0:HD]
  mask = xa_ref[:, HD:HD + 1]
  mean = num / jnp.maximum(cnt, 1.0)
  h = (jnp.dot(mean, wl_ref[...], preferred_element_type=jnp.float32)
       + bl_ref[...]
       + jnp.dot(x, wr_ref[...], preferred_element_type=jnp.float32))
  out_ref[...] = jnp.maximum(h, 0.0) * mask


def _conv(parts, xa, wl, bl, wr):
  grid = NP // _BR
  return pl.pallas_call(
      _conv_body,
      grid=(grid,),
      in_specs=[
          pl.BlockSpec((NC, _BR, WA), lambda i: (0, i, 0)),
          pl.BlockSpec((_BR, WA), lambda i: (i, 0)),
          pl.BlockSpec((HD, HD), lambda i: (0, 0)),
          pl.BlockSpec((1, HD), lambda i: (0, 0)),
          pl.BlockSpec((HD, HD), lambda i: (0, 0)),
      ],
      out_specs=pl.BlockSpec((_BR, HD), lambda i: (i, 0)),
      out_shape=jax.ShapeDtypeStruct((NP, HD), jnp.float32),
      name="sageconv",
  )(parts, xa, wl, bl.reshape(1, HD), wr)


def _score_body(parts_ref, x_ref, mask_ref, wrel_ref, brel_ref, wroot_ref,
                out_ref):
  agg = parts_ref[0] + parts_ref[1]
  pre = (jnp.dot(agg, wrel_ref[...], preferred_element_type=jnp.float32)
         + jnp.dot(x_ref[...], wroot_ref[...],
                   preferred_element_type=jnp.float32)
         + brel_ref[0, 0])
  mask = mask_ref[:, HD:HD + 1]
  out_ref[...] = jnp.where(mask > 0, jnp.tanh(pre), -3.0)


def _scores(parts, x, xa_prev, wrel, brel, wroot):
  grid = NP // _BR
  return pl.pallas_call(
      _score_body,
      grid=(grid,),
      in_specs=[
          pl.BlockSpec((NC, _BR, HD), lambda i: (0, i, 0)),
          pl.BlockSpec((_BR, HD), lambda i: (i, 0)),
          pl.BlockSpec((_BR, WA), lambda i: (i, 0)),
          pl.BlockSpec((HD, 1), lambda i: (0, 0)),
          pl.BlockSpec((1, 1), lambda i: (0, 0)),
          pl.BlockSpec((HD, 1), lambda i: (0, 0)),
      ],
      out_specs=pl.BlockSpec((_BR, 1), lambda i: (i, 0)),
      out_shape=jax.ShapeDtypeStruct((NP, 1), jnp.float32),
      name="sag_score",
  )(parts, x, xa_prev, wrel, brel.reshape(1, 1), wroot)


def _sortable(u):
  neg = (u >> jnp.uint32(31)) != jnp.uint32(0)
  return jnp.where(neg, ~u, u | jnp.uint32(0x80000000))


def _topk_body(kk, s_ref, x_ref, ident_ref, xa_ref, pooled_ref):
  s_col = s_ref[...]
  ident = ident_ref[...]
  dn = (((0,), (0,)), ((), ()))
  pieces = []
  for r in range(NP // 128):
    blk = s_col[r * 128:(r + 1) * 128, :]
    pieces.append(lax.dot_general(blk, ident, dn,
                                  preferred_element_type=jnp.float32))
  s_row = jnp.concatenate(pieces, axis=1)

  u_row = _sortable(lax.bitcast_convert_type(s_row, jnp.uint32))

  def count_ge(t):
    return jnp.sum((u_row >= t).astype(jnp.int32))

  def tstep(i, t):
    cand = t | (jnp.uint32(1) << (jnp.uint32(31) - i.astype(jnp.uint32)))
    return jnp.where(count_ge(cand) >= kk, cand, t)
  tthr = lax.fori_loop(0, 32, tstep, jnp.uint32(0))

  c_gt = jnp.sum((u_row > tthr).astype(jnp.int32))
  jtie = kk - c_gt
  idx_row = lax.broadcasted_iota(jnp.int32, (1, NP), 1)
  tie_row = (u_row == tthr)

  def pstep(i, p):
    cand = p + (jnp.int32(1) << (jnp.int32(13) - i))
    cnt = jnp.sum((tie_row & (idx_row < cand)).astype(jnp.int32))
    return jnp.where(cnt < jtie, cand, p)
  pcut = lax.fori_loop(0, 14, pstep, jnp.int32(0))

  u_col = _sortable(lax.bitcast_convert_type(s_col, jnp.uint32))
  idx_col = lax.broadcasted_iota(jnp.int32, (NP, 1), 0)
  sel = (u_col > tthr) | ((u_col == tthr) & (idx_col <= pcut))
  scale = jnp.where(sel, s_col, 0.0)
  x_out = x_ref[...] * scale

  xa_ref[...] = jnp.zeros((NP, WA), jnp.float32)
  xa_ref[:, 0:HD] = x_out
  colm = lax.broadcasted_iota(jnp.int32, (NP, WA - HD), 1)
  xa_ref[:, HD:WA] = jnp.where(
      (colm == 0) & sel, 1.0, 0.0)

  gmax = jnp.max(jnp.where(sel, x_out, NEG_INF), axis=0, keepdims=True)
  gmean = jnp.sum(x_out, axis=0, keepdims=True) * jnp.float32(1.0 / kk)
  pooled_ref[...] = jnp.concatenate([gmax, gmean], axis=1)


def _topk(kk, s, x):
  ident = jnp.eye(128, dtype=jnp.float32)
  return pl.pallas_call(
      functools.partial(_topk_body, kk),
      out_shape=(
          jax.ShapeDtypeStruct((NP, WA), jnp.float32),
          jax.ShapeDtypeStruct((1, 2 * HD), jnp.float32),
      ),
      name="sag_topk",
  )(s, x, ident)


def _head_body(p0_ref, p1_ref, p2_ref, w1_ref, b1_ref, w2_ref, b2_ref,
               gw_ref, gb_ref, hw_ref, hb_ref, f_ref, grade_ref, haz_ref):
  p = p0_ref[...] + p1_ref[...] + p2_ref[...]
  h = jnp.maximum(
      jnp.dot(p, w1_ref[...], preferred_element_type=jnp.float32)
      + b1_ref[...], 0.0)
  f = jnp.maximum(
      jnp.dot(h, w2_ref[...], preferred_element_type=jnp.float32)
      + b2_ref[...], 0.0)
  g = jnp.dot(f, gw_ref[...], preferred_element_type=jnp.float32) + gb_ref[...]
  gm = jnp.max(g, axis=1, keepdims=True)
  grade = g - gm - jnp.log(jnp.sum(jnp.exp(g - gm), axis=1, keepdims=True))
  z = jnp.dot(f, hw_ref[...], preferred_element_type=jnp.float32) + hb_ref[...]
  haz = (1.0 / (1.0 + jnp.exp(-z))) * 6.0 - 3.0
  f_ref[...] = f
  grade_ref[...] = grade
  haz_ref[...] = haz


def _head(p0, p1, p2, w1, b1, w2, b2, gw, gb, hw, hb):
  return pl.pallas_call(
      _head_body,
      out_shape=(
          jax.ShapeDtypeStruct((1, 32), jnp.float32),
          jax.ShapeDtypeStruct((1, 3), jnp.float32),
          jax.ShapeDtypeStruct((1, 1), jnp.float32),
      ),
      name="mlp_head",
  )(p0, p1, p2, w1, b1.reshape(1, -1), w2, b2.reshape(1, -1),
    gw, gb.reshape(1, -1), hw, hb.reshape(1, -1))


def kernel(x, edge_attr,
           conv0_Wl, conv0_bl, conv0_Wr, pool0_Wrel, pool0_brel, pool0_Wroot,
           conv1_Wl, conv1_bl, conv1_Wr, pool1_Wrel, pool1_brel, pool1_Wroot,
           conv2_Wl, conv2_bl, conv2_Wr, pool2_Wrel, pool2_brel, pool2_Wroot,
           enc_W1, enc_b1, enc_W2, enc_b2, grade_W, grade_b, haz_W, haz_b,
           edge_index, batch, pat_idxs):
  src = edge_index[0]
  dst = edge_index[1]
  pad = EP - NE
  padidx = jnp.arange(pad, dtype=jnp.int32)
  src_p = jnp.concatenate([src, NNODES + (padidx % 64)])
  dst_p = jnp.concatenate([dst, NNODES + 64 + (padidx % 64)])

  convs = [(conv0_Wl, conv0_bl, conv0_Wr),
           (conv1_Wl, conv1_bl, conv1_Wr),
           (conv2_Wl, conv2_bl, conv2_Wr)]
  pools = [(pool0_Wrel, pool0_brel, pool0_Wroot),
           (pool1_Wrel, pool1_brel, pool1_Wroot),
           (pool2_Wrel, pool2_brel, pool2_Wroot)]
  kks = [2000, 400, 80]

  xa = _normalize(x)
  pooled = []
  for (wl, bl, wr), (wrel, brel, wroot), kk in zip(convs, pools, kks):
    parts = _segsum_aug(xa, src_p, dst_p)
    x_new = _conv(parts, xa, wl, bl, wr)
    aparts = _segsum_feat(x_new, src_p, dst_p)
    s = _scores(aparts, x_new, xa, wrel, brel, wroot)
    xa, pld = _topk(kk, s, x_new)
    pooled.append(pld)

  return _head(pooled[0], pooled[1], pooled[2],
               enc_W1, enc_b1, enc_W2, enc_b2,
               grade_W, grade_b, haz_W, haz_b)


# speedup vs baseline: 17.7904x; 1.6367x over previous
import functools

import jax
import jax.numpy as jnp
from jax import lax
from jax.experimental import pallas as pl
from jax.experimental.pallas import tpu as pltpu
from jax.experimental.pallas import tpu_sc as plsc

NNODES = 10000
NP = 10240
NE = 320000
HD = 128
WA = 144
NC = 2
NS = 16
NW = NC * NS
CH = 64
CPW = 160
NSTG = 2
CPS = CPW // NSTG
EP = NW * CH * CPW
RPT = NP // NS
NEG_INF = float("-inf")


@functools.cache
def _make_segsum(width):
  mesh = plsc.VectorSubcoreMesh(core_axis_name="c", subcore_axis_name="s")

  def body(table, src, dst, out, srcb, dstb, rowsb, acc, sem0, sem1):
    cid = lax.axis_index("c")
    sid = lax.axis_index("s")
    wid = cid * NS + sid
    zero16 = jnp.zeros((16,), jnp.float32)
    sems = (sem0, sem1)

    def zrow(i, carry):
      for j in range(width // 16):
        rowsb[0, i, pl.ds(j * 16, 16)] = zero16
      return carry
    lax.fori_loop(0, CH, zrow, 0, unroll=False)
    for j in range(RPT // CH):
      pltpu.sync_copy(rowsb.at[0], acc.at[pl.ds(sid * RPT + j * CH, CH)])
    plsc.subcore_barrier()

    for stg in range(NSTG):
      base = wid * CPW + stg * CPS
      pltpu.sync_copy(src.at[pl.ds(base, CPS)], srcb)
      pltpu.sync_copy(dst.at[pl.ds(base, CPS)], dstb)
      pltpu.async_copy(table.at[srcb.at[0]], rowsb.at[0], sem0)
      pltpu.async_copy(table.at[srcb.at[1]], rowsb.at[1], sem1)

      def step(g, carry):
        for b in range(2):
          i = 2 * g + b
          pltpu.make_async_copy(table.at[srcb.at[0]], rowsb.at[b],
                                sems[b]).wait()
          pltpu.sync_copy(rowsb.at[b], acc.at[dstb.at[i]], add=True)
          @pl.when(i + 2 < CPS)
          def _():
            pltpu.async_copy(table.at[srcb.at[i + 2]], rowsb.at[b], sems[b])
        return carry
      lax.fori_loop(0, CPS // 2, step, 0, unroll=False)
    plsc.subcore_barrier()
    pltpu.sync_copy(acc.at[pl.ds(sid * RPT, RPT)],
                    out.at[cid, pl.ds(sid * RPT, RPT)])

  return pl.kernel(
      body,
      out_type=jax.ShapeDtypeStruct((NC, NP, width), jnp.float32),
      mesh=mesh,
      scratch_types=[
          pltpu.VMEM((CPS, CH), jnp.int32),
          pltpu.VMEM((CPS, CH), jnp.int32),
          pltpu.VMEM((2, CH, width), jnp.float32),
          pltpu.VMEM_SHARED((NP, width), jnp.float32),
          pltpu.SemaphoreType.DMA,
          pltpu.SemaphoreType.DMA,
      ],
      compiler_params=pltpu.CompilerParams(use_tc_tiling_on_sc=False),
      name=f"segsum{width}",
  )


def _segsum_aug(table, src, dst):
  return _make_segsum(WA)(table, src, dst)


def _segsum_feat(table, src, dst):
  return _make_segsum(HD)(table, src, dst)


def _norm_body(x_ref, out_ref):
  x = x_ref[...]
  m = jnp.max(x, axis=0, keepdims=True)
  col = lax.broadcasted_iota(jnp.int32, (1, HD), 1)
  xn = jnp.where(col < 12, x / m, x)
  out_ref[...] = jnp.zeros((NP, WA), jnp.float32)
  out_ref[0:NNODES, 0:HD] = xn
  col2 = lax.broadcasted_iota(jnp.int32, (NNODES, WA - HD), 1)
  out_ref[0:NNODES, HD:WA] = jnp.where(col2 == 0, 1.0, 0.0)


def _normalize(x):
  return pl.pallas_call(
      _norm_body,
      out_shape=jax.ShapeDtypeStruct((NP, WA), jnp.float32),
      name="normalize",
  )(x)


_BR = 1024


def _conv_body(parts_ref, xa_ref, wl_ref, bl_ref, wr_ref, out_ref):
  num = parts_ref[0, :, 0:HD] + parts_ref[1, :, 0:HD]
  cnt = parts_ref[0, :, HD:HD + 1] + parts_ref[1, :, HD:HD + 1]
  x = xa_ref[:, 0:HD]
  mask = xa_ref[:, HD:HD + 1]
  mean = num / jnp.maximum(cnt, 1.0)
  h = (jnp.dot(mean, wl_ref[...], preferred_element_type=jnp.float32)
       + bl_ref[...]
       + jnp.dot(x, wr_ref[...], preferred_element_type=jnp.float32))
  out_ref[...] = jnp.maximum(h, 0.0) * mask


def _conv(parts, xa, wl, bl, wr):
  grid = NP // _BR
  return pl.pallas_call(
      _conv_body,
      grid=(grid,),
      in_specs=[
          pl.BlockSpec((NC, _BR, WA), lambda i: (0, i, 0)),
          pl.BlockSpec((_BR, WA), lambda i: (i, 0)),
          pl.BlockSpec((HD, HD), lambda i: (0, 0)),
          pl.BlockSpec((1, HD), lambda i: (0, 0)),
          pl.BlockSpec((HD, HD), lambda i: (0, 0)),
      ],
      out_specs=pl.BlockSpec((_BR, HD), lambda i: (i, 0)),
      out_shape=jax.ShapeDtypeStruct((NP, HD), jnp.float32),
      name="sageconv",
  )(parts, xa, wl, bl.reshape(1, HD), wr)


def _score_body(parts_ref, x_ref, mask_ref, wrel_ref, brel_ref, wroot_ref,
                out_ref):
  agg = parts_ref[0] + parts_ref[1]
  pre = (jnp.dot(agg, wrel_ref[...], preferred_element_type=jnp.float32)
         + jnp.dot(x_ref[...], wroot_ref[...],
                   preferred_element_type=jnp.float32)
         + brel_ref[0, 0])
  mask = mask_ref[:, HD:HD + 1]
  out_ref[...] = jnp.where(mask > 0, jnp.tanh(pre), -3.0)


def _scores(parts, x, xa_prev, wrel, brel, wroot):
  grid = NP // _BR
  return pl.pallas_call(
      _score_body,
      grid=(grid,),
      in_specs=[
          pl.BlockSpec((NC, _BR, HD), lambda i: (0, i, 0)),
          pl.BlockSpec((_BR, HD), lambda i: (i, 0)),
          pl.BlockSpec((_BR, WA), lambda i: (i, 0)),
          pl.BlockSpec((HD, 1), lambda i: (0, 0)),
          pl.BlockSpec((1, 1), lambda i: (0, 0)),
          pl.BlockSpec((HD, 1), lambda i: (0, 0)),
      ],
      out_specs=pl.BlockSpec((_BR, 1), lambda i: (i, 0)),
      out_shape=jax.ShapeDtypeStruct((NP, 1), jnp.float32),
      name="sag_score",
  )(parts, x, xa_prev, wrel, brel.reshape(1, 1), wroot)


def _sortable(u):
  neg = (u >> jnp.uint32(31)) != jnp.uint32(0)
  return jnp.where(neg, ~u, u | jnp.uint32(0x80000000))


def _topk_body(kk, s_ref, x_ref, ident_ref, xa_ref, pooled_ref):
  s_col = s_ref[...]
  ident = ident_ref[...]
  dn = (((0,), (0,)), ((), ()))
  pieces = []
  for r in range(NP // 128):
    blk = s_col[r * 128:(r + 1) * 128, :]
    pieces.append(lax.dot_general(blk, ident, dn,
                                  preferred_element_type=jnp.float32))
  s_row = jnp.concatenate(pieces, axis=1)

  u_row = _sortable(lax.bitcast_convert_type(s_row, jnp.uint32))

  def count_ge(t):
    return jnp.sum((u_row >= t).astype(jnp.int32))

  def tstep(i, t):
    cand = t | (jnp.uint32(1) << (jnp.uint32(31) - i.astype(jnp.uint32)))
    return jnp.where(count_ge(cand) >= kk, cand, t)
  tthr = lax.fori_loop(0, 32, tstep, jnp.uint32(0))

  c_gt = jnp.sum((u_row > tthr).astype(jnp.int32))
  jtie = kk - c_gt
  idx_row = lax.broadcasted_iota(jnp.int32, (1, NP), 1)
  tie_row = (u_row == tthr)

  def pstep(i, p):
    cand = p + (jnp.int32(1) << (jnp.int32(13) - i))
    cnt = jnp.sum((tie_row & (idx_row < cand)).astype(jnp.int32))
    return jnp.where(cnt < jtie, cand, p)
  pcut = lax.fori_loop(0, 14, pstep, jnp.int32(0))

  u_col = _sortable(lax.bitcast_convert_type(s_col, jnp.uint32))
  idx_col = lax.broadcasted_iota(jnp.int32, (NP, 1), 0)
  sel = (u_col > tthr) | ((u_col == tthr) & (idx_col <= pcut))
  scale = jnp.where(sel, s_col, 0.0)
  x_out = x_ref[...] * scale

  xa_ref[...] = jnp.zeros((NP, WA), jnp.float32)
  xa_ref[:, 0:HD] = x_out
  colm = lax.broadcasted_iota(jnp.int32, (NP, WA - HD), 1)
  xa_ref[:, HD:WA] = jnp.where(
      (colm == 0) & sel, 1.0, 0.0)

  gmax = jnp.max(jnp.where(sel, x_out, NEG_INF), axis=0, keepdims=True)
  gmean = jnp.sum(x_out, axis=0, keepdims=True) * jnp.float32(1.0 / kk)
  pooled_ref[...] = jnp.concatenate([gmax, gmean], axis=1)


def _topk(kk, s, x):
  ident = jnp.eye(128, dtype=jnp.float32)
  return pl.pallas_call(
      functools.partial(_topk_body, kk),
      out_shape=(
          jax.ShapeDtypeStruct((NP, WA), jnp.float32),
          jax.ShapeDtypeStruct((1, 2 * HD), jnp.float32),
      ),
      name="sag_topk",
  )(s, x, ident)


def _head_body(p0_ref, p1_ref, p2_ref, w1_ref, b1_ref, w2_ref, b2_ref,
               gw_ref, gb_ref, hw_ref, hb_ref, f_ref, grade_ref, haz_ref):
  p = p0_ref[...] + p1_ref[...] + p2_ref[...]
  h = jnp.maximum(
      jnp.dot(p, w1_ref[...], preferred_element_type=jnp.float32)
      + b1_ref[...], 0.0)
  f = jnp.maximum(
      jnp.dot(h, w2_ref[...], preferred_element_type=jnp.float32)
      + b2_ref[...], 0.0)
  g = jnp.dot(f, gw_ref[...], preferred_element_type=jnp.float32) + gb_ref[...]
  gm = jnp.max(g, axis=1, keepdims=True)
  grade = g - gm - jnp.log(jnp.sum(jnp.exp(g - gm), axis=1, keepdims=True))
  z = jnp.dot(f, hw_ref[...], preferred_element_type=jnp.float32) + hb_ref[...]
  haz = (1.0 / (1.0 + jnp.exp(-z))) * 6.0 - 3.0
  f_ref[...] = f
  grade_ref[...] = grade
  haz_ref[...] = haz


def _head(p0, p1, p2, w1, b1, w2, b2, gw, gb, hw, hb):
  return pl.pallas_call(
      _head_body,
      out_shape=(
          jax.ShapeDtypeStruct((1, 32), jnp.float32),
          jax.ShapeDtypeStruct((1, 3), jnp.float32),
          jax.ShapeDtypeStruct((1, 1), jnp.float32),
      ),
      name="mlp_head",
  )(p0, p1, p2, w1, b1.reshape(1, -1), w2, b2.reshape(1, -1),
    gw, gb.reshape(1, -1), hw, hb.reshape(1, -1))


def kernel(x, edge_attr,
           conv0_Wl, conv0_bl, conv0_Wr, pool0_Wrel, pool0_brel, pool0_Wroot,
           conv1_Wl, conv1_bl, conv1_Wr, pool1_Wrel, pool1_brel, pool1_Wroot,
           conv2_Wl, conv2_bl, conv2_Wr, pool2_Wrel, pool2_brel, pool2_Wroot,
           enc_W1, enc_b1, enc_W2, enc_b2, grade_W, grade_b, haz_W, haz_b,
           edge_index, batch, pat_idxs):
  src = edge_index[0]
  dst = edge_index[1]
  pad = EP - NE
  padidx = jnp.arange(pad, dtype=jnp.int32)
  src_p = jnp.concatenate([src, NNODES + (padidx % 64)]).reshape(NW * CPW, CH)
  dst_p = jnp.concatenate([dst, NNODES + 64 + (padidx % 64)]
                          ).reshape(NW * CPW, CH)

  convs = [(conv0_Wl, conv0_bl, conv0_Wr),
           (conv1_Wl, conv1_bl, conv1_Wr),
           (conv2_Wl, conv2_bl, conv2_Wr)]
  pools = [(pool0_Wrel, pool0_brel, pool0_Wroot),
           (pool1_Wrel, pool1_brel, pool1_Wroot),
           (pool2_Wrel, pool2_brel, pool2_Wroot)]
  kks = [2000, 400, 80]

  xa = _normalize(x)
  pooled = []
  for (wl, bl, wr), (wrel, brel, wroot), kk in zip(convs, pools, kks):
    parts = _segsum_aug(xa, src_p, dst_p)
    x_new = _conv(parts, xa, wl, bl, wr)
    aparts = _segsum_feat(x_new, src_p, dst_p)
    s = _scores(aparts, x_new, xa, wrel, brel, wroot)
    xa, pld = _topk(kk, s, x_new)
    pooled.append(pld)

  return _head(pooled[0], pooled[1], pooled[2],
               enc_W1, enc_b1, enc_W2, enc_b2,
               grade_W, grade_b, haz_W, haz_b)
